# baseline (speedup 1.0000x reference)
import math
import functools

import jax
import jax.numpy as jnp
from jax.experimental import pallas as pl
from jax.experimental.pallas import tpu as pltpu

N_EDGES = 160000
E_EDGE = 256
E_QIN = 32
E_SBF = 32
E_RBF = 16
E_CBF = 16
E_QOUT = 32
NSPH = 8
KMAX = 8

BLK_E = 1000
BLK_T = 2000
BLK_D = 800
BLK_F = 1000


def _stage1_body(m_ref, br_ref, wdb_ref, wrbf_ref, wdown_ref, out_ref):
    t = jnp.dot(m_ref[...], wdb_ref[...], preferred_element_type=jnp.float32)
    rb = jnp.dot(br_ref[...], wrbf_ref[...], preferred_element_type=jnp.float32)
    out_ref[...] = jnp.dot(t * rb, wdown_ref[...], preferred_element_type=jnp.float32)


def _stage1(m, bases_rad, W_db, W_rbf, W_down):
    n = m.shape[0]
    return pl.pallas_call(
        _stage1_body,
        grid=(n // BLK_E,),
        in_specs=[
            pl.BlockSpec((BLK_E, E_EDGE), lambda i: (i, 0)),
            pl.BlockSpec((BLK_E, E_RBF), lambda i: (i, 0)),
            pl.BlockSpec((E_EDGE, E_EDGE), lambda i: (0, 0)),
            pl.BlockSpec((E_RBF, E_EDGE), lambda i: (0, 0)),
            pl.BlockSpec((E_EDGE, E_QIN), lambda i: (0, 0)),
        ],
        out_specs=pl.BlockSpec((BLK_E, E_QIN), lambda i: (i, 0)),
        out_shape=jax.ShapeDtypeStruct((n, E_QIN), jnp.float32),
    )(m, bases_rad, W_db, W_rbf, W_down)


def _cb_body(bc_ref, wcbf_ref, out_ref):
    out_ref[...] = jnp.dot(bc_ref[...], wcbf_ref[...], preferred_element_type=jnp.float32)


def _cb(bases_cir, W_cbf):
    n = bases_cir.shape[0]
    return pl.pallas_call(
        _cb_body,
        grid=(n // BLK_T,),
        in_specs=[
            pl.BlockSpec((BLK_T, E_CBF), lambda i: (i, 0)),
            pl.BlockSpec((E_CBF, E_QIN), lambda i: (0, 0)),
        ],
        out_specs=pl.BlockSpec((BLK_T, E_QIN), lambda i: (i, 0)),
        out_shape=jax.ShapeDtypeStruct((n, E_QIN), jnp.float32),
    )(bases_cir, W_cbf)


def _rep_lanes(a, rep):
    B, L = a.shape
    return jnp.broadcast_to(a[:, :, None], (B, L, rep)).reshape(B, L * rep)


def _tile_lanes(a, rep):
    B, L = a.shape
    return jnp.broadcast_to(a[:, None, :], (B, rep, L)).reshape(B, rep * L)


def _stageD_body(mp_ref, ss_ref, w1r_ref, wb_ref, out_ref):
    B = mp_ref.shape[0]
    smf = jnp.zeros((B, NSPH * E_QIN), jnp.float32)
    for k in range(KMAX):
        ssk = ss_ref[:, k * NSPH:(k + 1) * NSPH]
        mpk = mp_ref[:, k * E_QIN:(k + 1) * E_QIN]
        smf = smf + _rep_lanes(ssk, E_QIN) * _tile_lanes(mpk, NSPH)
    r = jnp.zeros((B, E_SBF * E_QIN), jnp.float32)
    for n in range(NSPH):
        w1n = w1r_ref[:, n * E_SBF:(n + 1) * E_SBF]
        smn = smf[:, n * E_QIN:(n + 1) * E_QIN]
        r = r + _rep_lanes(w1n, E_QIN) * _tile_lanes(smn, E_SBF)
    out_ref[...] = jnp.dot(r, wb_ref[...], preferred_element_type=jnp.float32)


def _stageD(mp_flat, ss_flat, w1r_flat, W_bil):
    n = mp_flat.shape[0]
    return pl.pallas_call(
        _stageD_body,
        grid=(n // BLK_D,),
        in_specs=[
            pl.BlockSpec((BLK_D, KMAX * E_QIN), lambda i: (i, 0)),
            pl.BlockSpec((BLK_D, KMAX * NSPH), lambda i: (i, 0)),
            pl.BlockSpec((BLK_D, NSPH * E_SBF), lambda i: (i, 0)),
            pl.BlockSpec((E_SBF * E_QIN, E_QOUT), lambda i: (0, 0)),
        ],
        out_specs=pl.BlockSpec((BLK_D, E_QOUT), lambda i: (i, 0)),
        out_shape=jax.ShapeDtypeStruct((n, E_QOUT), jnp.float32),
    )(mp_flat, ss_flat, w1r_flat, W_bil)


def _final_body(x_ref, xg_ref, wca_ref, wac_ref, out_ref):
    inv_sqrt_2 = 1.0 / math.sqrt(2.0)
    a = jnp.dot(x_ref[...], wca_ref[...], preferred_element_type=jnp.float32)
    b = jnp.dot(xg_ref[...], wac_ref[...], preferred_element_type=jnp.float32)
    out_ref[...] = (a + b) * inv_sqrt_2


def _final(x, xg, W_up_ca, W_up_ac):
    n = x.shape[0]
    return pl.pallas_call(
        _final_body,
        grid=(n // BLK_F,),
        in_specs=[
            pl.BlockSpec((BLK_F, E_QOUT), lambda i: (i, 0)),
            pl.BlockSpec((BLK_F, E_QOUT), lambda i: (i, 0)),
            pl.BlockSpec((E_QOUT, E_EDGE), lambda i: (0, 0)),
            pl.BlockSpec((E_QOUT, E_EDGE), lambda i: (0, 0)),
        ],
        out_specs=pl.BlockSpec((BLK_F, E_EDGE), lambda i: (i, 0)),
        out_shape=jax.ShapeDtypeStruct((n, E_EDGE), jnp.float32),
    )(x, xg, W_up_ca, W_up_ac)


def kernel(m, bases_rad, bases_cir, sph_rbf_W1, sph_sph, idx_triplet_in_in,
           idx_trip_in_to_quad, idx_out, idx_out_agg, id_swap,
           W_db, W_rbf, W_cbf, W_down, W_bil, W_up_ca, W_up_ac):
    x1 = _stage1(m, bases_rad, W_db, W_rbf, W_down)
    cb = _cb(bases_cir, W_cbf)
    x_db = jnp.take(x1, idx_triplet_in_in, axis=0) * cb
    x_db = jnp.take(x_db, idx_trip_in_to_quad, axis=0)
    nE = sph_rbf_W1.shape[0]
    m_pad = jnp.zeros((nE, KMAX, E_QIN), jnp.float32).at[idx_out, idx_out_agg].set(x_db)
    mp_flat = m_pad.reshape(nE, KMAX * E_QIN)
    ss_flat = sph_sph.reshape(nE, KMAX * NSPH)
    w1r_flat = jnp.transpose(sph_rbf_W1, (0, 2, 1)).reshape(nE, NSPH * E_SBF)
    x = _stageD(mp_flat, ss_flat, w1r_flat, W_bil)
    xg = jnp.take(x, id_swap, axis=0)
    return _final(x, xg, W_up_ca, W_up_ac)


# speedup vs baseline: 1.7807x; 1.7807x over previous
import math
import functools

import jax
import jax.numpy as jnp
from jax.experimental import pallas as pl
from jax.experimental.pallas import tpu as pltpu

N_EDGES = 160000
E_EDGE = 256
E_QIN = 32
E_SBF = 32
E_RBF = 16
E_CBF = 16
E_QOUT = 32
NSPH = 8
KMAX = 8

BLK_E = 1000
BLK_T = 2000
BLK_D = 800
BLK_F = 1000


def _stage1_body(m_ref, br_ref, wdb_ref, wrbf_ref, wdown_ref, out_ref):
    t = jnp.dot(m_ref[...], wdb_ref[...], preferred_element_type=jnp.float32)
    rb = jnp.dot(br_ref[...], wrbf_ref[...], preferred_element_type=jnp.float32)
    out_ref[...] = jnp.dot(t * rb, wdown_ref[...], preferred_element_type=jnp.float32)


def _stage1(m, bases_rad, W_db, W_rbf, W_down):
    n = m.shape[0]
    return pl.pallas_call(
        _stage1_body,
        grid=(n // BLK_E,),
        in_specs=[
            pl.BlockSpec((BLK_E, E_EDGE), lambda i: (i, 0)),
            pl.BlockSpec((BLK_E, E_RBF), lambda i: (i, 0)),
            pl.BlockSpec((E_EDGE, E_EDGE), lambda i: (0, 0)),
            pl.BlockSpec((E_RBF, E_EDGE), lambda i: (0, 0)),
            pl.BlockSpec((E_EDGE, E_QIN), lambda i: (0, 0)),
        ],
        out_specs=pl.BlockSpec((BLK_E, E_QIN), lambda i: (i, 0)),
        out_shape=jax.ShapeDtypeStruct((n, E_QIN), jnp.float32),
    )(m, bases_rad, W_db, W_rbf, W_down)


def _cb_body(bc_ref, wcbf_ref, out_ref):
    out_ref[...] = jnp.dot(bc_ref[...], wcbf_ref[...], preferred_element_type=jnp.float32)


def _cb(bases_cir, W_cbf):
    n = bases_cir.shape[0]
    return pl.pallas_call(
        _cb_body,
        grid=(n // BLK_T,),
        in_specs=[
            pl.BlockSpec((BLK_T, E_CBF), lambda i: (i, 0)),
            pl.BlockSpec((E_CBF, E_QIN), lambda i: (0, 0)),
        ],
        out_specs=pl.BlockSpec((BLK_T, E_QIN), lambda i: (i, 0)),
        out_shape=jax.ShapeDtypeStruct((n, E_QIN), jnp.float32),
    )(bases_cir, W_cbf)


def _stageD_body(mp_ref, ss_ref, w1r_ref, wb_ref, r8_ref, t8_ref, r32_ref,
                 t32_ref, out_ref):
    B = mp_ref.shape[0]
    f32 = jnp.float32
    smf = jnp.zeros((B, NSPH * E_QIN), f32)
    for k in range(KMAX):
        ssk = ss_ref[:, k * NSPH:(k + 1) * NSPH]
        mpk = mp_ref[:, k * E_QIN:(k + 1) * E_QIN]
        smf = smf + (jnp.dot(ssk, r8_ref[...], preferred_element_type=f32) *
                     jnp.dot(mpk, t8_ref[...], preferred_element_type=f32))
    r = jnp.zeros((B, E_SBF * E_QIN), f32)
    for n in range(NSPH):
        w1n = w1r_ref[:, n * E_SBF:(n + 1) * E_SBF]
        smn = smf[:, n * E_QIN:(n + 1) * E_QIN]
        r = r + (jnp.dot(w1n, r32_ref[...], preferred_element_type=f32) *
                 jnp.dot(smn, t32_ref[...], preferred_element_type=f32))
    out_ref[...] = jnp.dot(r, wb_ref[...], preferred_element_type=f32)


def _expander_mats():
    i8 = jnp.arange(NSPH)
    i32 = jnp.arange(E_QIN)
    j256 = jnp.arange(NSPH * E_QIN)
    j1024 = jnp.arange(E_SBF * E_QIN)
    r8 = (j256[None, :] // E_QIN == i8[:, None]).astype(jnp.float32)
    t8 = (j256[None, :] % E_QIN == i32[:, None]).astype(jnp.float32)
    r32 = (j1024[None, :] // E_QIN == i32[:, None]).astype(jnp.float32)
    t32 = (j1024[None, :] % E_QIN == i32[:, None]).astype(jnp.float32)
    return r8, t8, r32, t32


def _stageD(mp_flat, ss_flat, w1r_flat, W_bil):
    n = mp_flat.shape[0]
    r8, t8, r32, t32 = _expander_mats()
    return pl.pallas_call(
        _stageD_body,
        grid=(n // BLK_D,),
        in_specs=[
            pl.BlockSpec((BLK_D, KMAX * E_QIN), lambda i: (i, 0)),
            pl.BlockSpec((BLK_D, KMAX * NSPH), lambda i: (i, 0)),
            pl.BlockSpec((BLK_D, NSPH * E_SBF), lambda i: (i, 0)),
            pl.BlockSpec((E_SBF * E_QIN, E_QOUT), lambda i: (0, 0)),
            pl.BlockSpec((NSPH, NSPH * E_QIN), lambda i: (0, 0)),
            pl.BlockSpec((E_QIN, NSPH * E_QIN), lambda i: (0, 0)),
            pl.BlockSpec((E_QIN, E_SBF * E_QIN), lambda i: (0, 0)),
            pl.BlockSpec((E_QIN, E_SBF * E_QIN), lambda i: (0, 0)),
        ],
        out_specs=pl.BlockSpec((BLK_D, E_QOUT), lambda i: (i, 0)),
        out_shape=jax.ShapeDtypeStruct((n, E_QOUT), jnp.float32),
    )(mp_flat, ss_flat, w1r_flat, W_bil, r8, t8, r32, t32)


def _final_body(x_ref, xg_ref, wca_ref, wac_ref, out_ref):
    inv_sqrt_2 = 1.0 / math.sqrt(2.0)
    a = jnp.dot(x_ref[...], wca_ref[...], preferred_element_type=jnp.float32)
    b = jnp.dot(xg_ref[...], wac_ref[...], preferred_element_type=jnp.float32)
    out_ref[...] = (a + b) * inv_sqrt_2


def _final(x, xg, W_up_ca, W_up_ac):
    n = x.shape[0]
    return pl.pallas_call(
        _final_body,
        grid=(n // BLK_F,),
        in_specs=[
            pl.BlockSpec((BLK_F, E_QOUT), lambda i: (i, 0)),
            pl.BlockSpec((BLK_F, E_QOUT), lambda i: (i, 0)),
            pl.BlockSpec((E_QOUT, E_EDGE), lambda i: (0, 0)),
            pl.BlockSpec((E_QOUT, E_EDGE), lambda i: (0, 0)),
        ],
        out_specs=pl.BlockSpec((BLK_F, E_EDGE), lambda i: (i, 0)),
        out_shape=jax.ShapeDtypeStruct((n, E_EDGE), jnp.float32),
    )(x, xg, W_up_ca, W_up_ac)


def kernel(m, bases_rad, bases_cir, sph_rbf_W1, sph_sph, idx_triplet_in_in,
           idx_trip_in_to_quad, idx_out, idx_out_agg, id_swap,
           W_db, W_rbf, W_cbf, W_down, W_bil, W_up_ca, W_up_ac):
    x1 = _stage1(m, bases_rad, W_db, W_rbf, W_down)
    cb = _cb(bases_cir, W_cbf)
    x_db = jnp.take(x1, idx_triplet_in_in, axis=0) * cb
    x_db = jnp.take(x_db, idx_trip_in_to_quad, axis=0)
    nE = sph_rbf_W1.shape[0]
    m_pad = jnp.zeros((nE, KMAX, E_QIN), jnp.float32).at[idx_out, idx_out_agg].set(x_db)
    mp_flat = m_pad.reshape(nE, KMAX * E_QIN)
    ss_flat = sph_sph.reshape(nE, KMAX * NSPH)
    w1r_flat = jnp.transpose(sph_rbf_W1, (0, 2, 1)).reshape(nE, NSPH * E_SBF)
    x = _stageD(mp_flat, ss_flat, w1r_flat, W_bil)
    xg = jnp.take(x, id_swap, axis=0)
    return _final(x, xg, W_up_ca, W_up_ac)


# speedup vs baseline: 3.2193x; 1.8079x over previous
import math
import functools

import jax
import jax.numpy as jnp
from jax import lax
from jax.experimental import pallas as pl
from jax.experimental.pallas import tpu as pltpu, tpu_sc as plsc

N_EDGES = 160000
E_EDGE = 256
E_QIN = 32
E_SBF = 32
E_RBF = 16
E_CBF = 16
E_QOUT = 32
NSPH = 8
KMAX = 8

BLK_E = 1000
BLK_T = 2000
BLK_D = 800
BLK_F = 1000


def _stage1_body(m_ref, br_ref, wdb_ref, wrbf_ref, wdown_ref, out_ref):
    t = jnp.dot(m_ref[...], wdb_ref[...], preferred_element_type=jnp.float32)
    rb = jnp.dot(br_ref[...], wrbf_ref[...], preferred_element_type=jnp.float32)
    out_ref[...] = jnp.dot(t * rb, wdown_ref[...], preferred_element_type=jnp.float32)


def _stage1(m, bases_rad, W_db, W_rbf, W_down):
    n = m.shape[0]
    return pl.pallas_call(
        _stage1_body,
        grid=(n // BLK_E,),
        in_specs=[
            pl.BlockSpec((BLK_E, E_EDGE), lambda i: (i, 0)),
            pl.BlockSpec((BLK_E, E_RBF), lambda i: (i, 0)),
            pl.BlockSpec((E_EDGE, E_EDGE), lambda i: (0, 0)),
            pl.BlockSpec((E_RBF, E_EDGE), lambda i: (0, 0)),
            pl.BlockSpec((E_EDGE, E_QIN), lambda i: (0, 0)),
        ],
        out_specs=pl.BlockSpec((BLK_E, E_QIN), lambda i: (i, 0)),
        out_shape=jax.ShapeDtypeStruct((n, E_QIN), jnp.float32),
    )(m, bases_rad, W_db, W_rbf, W_down)


def _cb_body(bc_ref, wcbf_ref, out_ref):
    out_ref[...] = jnp.dot(bc_ref[...], wcbf_ref[...], preferred_element_type=jnp.float32)


def _cb(bases_cir, W_cbf):
    n = bases_cir.shape[0]
    return pl.pallas_call(
        _cb_body,
        grid=(n // BLK_T,),
        in_specs=[
            pl.BlockSpec((BLK_T, E_CBF), lambda i: (i, 0)),
            pl.BlockSpec((E_CBF, E_QIN), lambda i: (0, 0)),
        ],
        out_specs=pl.BlockSpec((BLK_T, E_QIN), lambda i: (i, 0)),
        out_shape=jax.ShapeDtypeStruct((n, E_QIN), jnp.float32),
    )(bases_cir, W_cbf)


def _stageD_body(mp_ref, ss_ref, w1r_ref, wb_ref, r8_ref, t8_ref, r32_ref,
                 t32_ref, out_ref):
    B = mp_ref.shape[0]
    f32 = jnp.float32
    smf = jnp.zeros((B, NSPH * E_QIN), f32)
    for k in range(KMAX):
        ssk = ss_ref[:, k * NSPH:(k + 1) * NSPH]
        mpk = mp_ref[:, k * E_QIN:(k + 1) * E_QIN]
        smf = smf + (jnp.dot(ssk, r8_ref[...], preferred_element_type=f32) *
                     jnp.dot(mpk, t8_ref[...], preferred_element_type=f32))
    r = jnp.zeros((B, E_SBF * E_QIN), f32)
    for n in range(NSPH):
        w1n = w1r_ref[:, n * E_SBF:(n + 1) * E_SBF]
        smn = smf[:, n * E_QIN:(n + 1) * E_QIN]
        r = r + (jnp.dot(w1n, r32_ref[...], preferred_element_type=f32) *
                 jnp.dot(smn, t32_ref[...], preferred_element_type=f32))
    out_ref[...] = jnp.dot(r, wb_ref[...], preferred_element_type=f32)


def _expander_mats():
    i8 = jnp.arange(NSPH)
    i32 = jnp.arange(E_QIN)
    j256 = jnp.arange(NSPH * E_QIN)
    j1024 = jnp.arange(E_SBF * E_QIN)
    r8 = (j256[None, :] // E_QIN == i8[:, None]).astype(jnp.float32)
    t8 = (j256[None, :] % E_QIN == i32[:, None]).astype(jnp.float32)
    r32 = (j1024[None, :] // E_QIN == i32[:, None]).astype(jnp.float32)
    t32 = (j1024[None, :] % E_QIN == i32[:, None]).astype(jnp.float32)
    return r8, t8, r32, t32


def _stageD(mp_flat, ss_flat, w1r_flat, W_bil):
    n = mp_flat.shape[0]
    r8, t8, r32, t32 = _expander_mats()
    return pl.pallas_call(
        _stageD_body,
        grid=(n // BLK_D,),
        in_specs=[
            pl.BlockSpec((BLK_D, KMAX * E_QIN), lambda i: (i, 0)),
            pl.BlockSpec((BLK_D, KMAX * NSPH), lambda i: (i, 0)),
            pl.BlockSpec((BLK_D, NSPH * E_SBF), lambda i: (i, 0)),
            pl.BlockSpec((E_SBF * E_QIN, E_QOUT), lambda i: (0, 0)),
            pl.BlockSpec((NSPH, NSPH * E_QIN), lambda i: (0, 0)),
            pl.BlockSpec((E_QIN, NSPH * E_QIN), lambda i: (0, 0)),
            pl.BlockSpec((E_QIN, E_SBF * E_QIN), lambda i: (0, 0)),
            pl.BlockSpec((E_QIN, E_SBF * E_QIN), lambda i: (0, 0)),
        ],
        out_specs=pl.BlockSpec((BLK_D, E_QOUT), lambda i: (i, 0)),
        out_shape=jax.ShapeDtypeStruct((n, E_QOUT), jnp.float32),
    )(mp_flat, ss_flat, w1r_flat, W_bil, r8, t8, r32, t32)


def _final_body(x_ref, xg_ref, wca_ref, wac_ref, out_ref):
    inv_sqrt_2 = 1.0 / math.sqrt(2.0)
    a = jnp.dot(x_ref[...], wca_ref[...], preferred_element_type=jnp.float32)
    b = jnp.dot(xg_ref[...], wac_ref[...], preferred_element_type=jnp.float32)
    out_ref[...] = (a + b) * inv_sqrt_2


def _final(x, xg, W_up_ca, W_up_ac):
    n = x.shape[0]
    return pl.pallas_call(
        _final_body,
        grid=(n // BLK_F,),
        in_specs=[
            pl.BlockSpec((BLK_F, E_QOUT), lambda i: (i, 0)),
            pl.BlockSpec((BLK_F, E_QOUT), lambda i: (i, 0)),
            pl.BlockSpec((E_QOUT, E_EDGE), lambda i: (0, 0)),
            pl.BlockSpec((E_QOUT, E_EDGE), lambda i: (0, 0)),
        ],
        out_specs=pl.BlockSpec((BLK_F, E_EDGE), lambda i: (i, 0)),
        out_shape=jax.ShapeDtypeStruct((n, E_EDGE), jnp.float32),
    )(x, xg, W_up_ca, W_up_ac)


N_QUAD = 640000
N_TRIP = 640000
NSLOTS = N_EDGES * KMAX
NW = 32
OWN = NSLOTS // NW
CHUNK = 16000
WIN = 400

_SC_PARAMS = pltpu.CompilerParams(use_tc_tiling_on_sc=False,
                                  needs_layout_passes=False)


def _sc_mesh():
    return plsc.VectorSubcoreMesh(core_axis_name="c", subcore_axis_name="s")


def _sc_build_mpad(idx_out, idx_out_agg, ttq_ext, tii_ext, x1z, cbz):
    @functools.partial(
        pl.kernel, mesh=_sc_mesh(),
        out_type=jax.ShapeDtypeStruct((NSLOTS, E_QIN), jnp.float32),
        compiler_params=_SC_PARAMS,
        scratch_types=[
            pltpu.VMEM((OWN,), jnp.int32),
            pltpu.VMEM((CHUNK,), jnp.int32),
            pltpu.VMEM((CHUNK,), jnp.int32),
            pltpu.VMEM((WIN,), jnp.int32),
            pltpu.VMEM((WIN,), jnp.int32),
            pltpu.VMEM((WIN,), jnp.int32),
            pltpu.VMEM((WIN, E_QIN), jnp.float32),
            pltpu.VMEM((WIN, E_QIN), jnp.float32),
            pltpu.VMEM((WIN, E_QIN), jnp.float32),
            pltpu.SemaphoreType.DMA,
        ],
    )
    def k(io_hbm, ia_hbm, ttq_hbm, tii_hbm, x1_hbm, cb_hbm, out_hbm,
          winner_v, io_v, ia_v, widx_v, t_v, e2_v, x1_v, cb_v, o_v, sem):
        wid = lax.axis_index("s") * 2 + lax.axis_index("c")
        lo = wid * OWN
        iota16 = lax.iota(jnp.int32, 16)
        zero16 = iota16 - iota16

        def zi(i, c):
            winner_v[pl.ds(i * 16, 16)] = zero16
            return c
        lax.fori_loop(0, OWN // 16, zi, 0)

        def chunk_body(ci, c):
            pltpu.sync_copy(io_hbm.at[pl.ds(ci * CHUNK, CHUNK)], io_v)
            pltpu.sync_copy(ia_hbm.at[pl.ds(ci * CHUNK, CHUNK)], ia_v)
            qbase = ci * CHUNK + 1

            def vb(i, c2):
                io = io_v[pl.ds(i * 16, 16)]
                ia = ia_v[pl.ds(i * 16, 16)]
                local = io * KMAX + ia - lo
                mask = (local >= 0) & (local < OWN)
                localc = jnp.where(mask, local, 0)
                q1 = (qbase + i * 16) + iota16
                plsc.store_scatter(winner_v, [localc], q1, mask=mask)
                return c2
            lax.fori_loop(0, CHUNK // 16, vb, 0)
            return c
        lax.fori_loop(0, N_QUAD // CHUNK, chunk_body, 0)

        def win_body(wi, c):
            woff = wi * WIN

            def tb(i, c2):
                wv = winner_v[pl.ds(woff + i * 16, 16)]
                widx_v[pl.ds(i * 16, 16)] = jnp.where(wv == 0, N_TRIP, wv - 1)
                return c2
            lax.fori_loop(0, WIN // 16, tb, 0)
            pltpu.async_copy(ttq_hbm.at[widx_v], t_v, sem).wait()
            pltpu.async_copy(tii_hbm.at[t_v], e2_v, sem).wait()
            pltpu.async_copy(x1_hbm.at[e2_v], x1_v, sem).wait()
            pltpu.async_copy(cb_hbm.at[t_v], cb_v, sem).wait()

            def mb(i, c2):
                o_v[i, pl.ds(0, 16)] = x1_v[i, pl.ds(0, 16)] * cb_v[i, pl.ds(0, 16)]
                o_v[i, pl.ds(16, 16)] = x1_v[i, pl.ds(16, 16)] * cb_v[i, pl.ds(16, 16)]
                return c2
            lax.fori_loop(0, WIN, mb, 0)
            pltpu.sync_copy(o_v, out_hbm.at[pl.ds(lo + woff, WIN)])
            return c
        lax.fori_loop(0, OWN // WIN, win_body, 0)

    return k(idx_out, idx_out_agg, ttq_ext, tii_ext, x1z, cbz)


GW = N_EDGES // NW
GWIN = 1000


def _sc_gather_rows(x, idx):
    @functools.partial(
        pl.kernel, mesh=_sc_mesh(),
        out_type=jax.ShapeDtypeStruct((N_EDGES, E_QOUT), jnp.float32),
        compiler_params=_SC_PARAMS,
        scratch_types=[
            pltpu.VMEM((GWIN,), jnp.int32),
            pltpu.VMEM((GWIN, E_QOUT), jnp.float32),
            pltpu.SemaphoreType.DMA,
        ],
    )
    def k(x_hbm, idx_hbm, out_hbm, idx_v, rows_v, sem):
        wid = lax.axis_index("s") * 2 + lax.axis_index("c")
        base = wid * GW

        def wb(i, c):
            off = base + i * GWIN
            pltpu.sync_copy(idx_hbm.at[pl.ds(off, GWIN)], idx_v)
            pltpu.async_copy(x_hbm.at[idx_v], rows_v, sem).wait()
            pltpu.sync_copy(rows_v, out_hbm.at[pl.ds(off, GWIN)])
            return c
        lax.fori_loop(0, GW // GWIN, wb, 0)

    return k(x, idx)


def kernel(m, bases_rad, bases_cir, sph_rbf_W1, sph_sph, idx_triplet_in_in,
           idx_trip_in_to_quad, idx_out, idx_out_agg, id_swap,
           W_db, W_rbf, W_cbf, W_down, W_bil, W_up_ca, W_up_ac):
    x1 = _stage1(m, bases_rad, W_db, W_rbf, W_down)
    cb = _cb(bases_cir, W_cbf)
    nE = sph_rbf_W1.shape[0]
    x1z = jnp.concatenate([x1, jnp.zeros((1, E_QIN), jnp.float32)], axis=0)
    cbz = jnp.concatenate([cb, jnp.zeros((1, E_QIN), jnp.float32)], axis=0)
    ttq_ext = jnp.concatenate(
        [idx_trip_in_to_quad.astype(jnp.int32),
         jnp.full((1,), N_TRIP, jnp.int32)])
    tii_ext = jnp.concatenate(
        [idx_triplet_in_in.astype(jnp.int32),
         jnp.full((1,), N_EDGES, jnp.int32)])
    mp = _sc_build_mpad(idx_out.astype(jnp.int32), idx_out_agg.astype(jnp.int32),
                        ttq_ext, tii_ext, x1z, cbz)
    mp_flat = mp.reshape(nE, KMAX * E_QIN)
    ss_flat = sph_sph.reshape(nE, KMAX * NSPH)
    w1r_flat = jnp.transpose(sph_rbf_W1, (0, 2, 1)).reshape(nE, NSPH * E_SBF)
    x = _stageD(mp_flat, ss_flat, w1r_flat, W_bil)
    xg = _sc_gather_rows(x, id_swap.astype(jnp.int32))
    return _final(x, xg, W_up_ca, W_up_ac)


# speedup vs baseline: 7.7556x; 2.4091x over previous
import math
import functools

import jax
import jax.numpy as jnp
from jax import lax
from jax.experimental import pallas as pl
from jax.experimental.pallas import tpu as pltpu, tpu_sc as plsc

N_EDGES = 160000
E_EDGE = 256
E_QIN = 32
E_SBF = 32
E_RBF = 16
E_CBF = 16
E_QOUT = 32
NSPH = 8
KMAX = 8

BLK_E = 1000
BLK_T = 2000
BLK_D = 800
BLK_F = 1000


def _stage1_body(m_ref, br_ref, wdb_ref, wrbf_ref, wdown_ref, out_ref):
    t = jnp.dot(m_ref[...], wdb_ref[...], preferred_element_type=jnp.float32)
    rb = jnp.dot(br_ref[...], wrbf_ref[...], preferred_element_type=jnp.float32)
    out_ref[...] = jnp.dot(t * rb, wdown_ref[...], preferred_element_type=jnp.float32)


def _stage1(m, bases_rad, W_db, W_rbf, W_down):
    n = m.shape[0]
    return pl.pallas_call(
        _stage1_body,
        grid=(n // BLK_E,),
        in_specs=[
            pl.BlockSpec((BLK_E, E_EDGE), lambda i: (i, 0)),
            pl.BlockSpec((BLK_E, E_RBF), lambda i: (i, 0)),
            pl.BlockSpec((E_EDGE, E_EDGE), lambda i: (0, 0)),
            pl.BlockSpec((E_RBF, E_EDGE), lambda i: (0, 0)),
            pl.BlockSpec((E_EDGE, E_QIN), lambda i: (0, 0)),
        ],
        out_specs=pl.BlockSpec((BLK_E, E_QIN), lambda i: (i, 0)),
        out_shape=jax.ShapeDtypeStruct((n, E_QIN), jnp.float32),
    )(m, bases_rad, W_db, W_rbf, W_down)


def _cb_body(bc_ref, wcbf_ref, out_ref):
    out_ref[...] = jnp.dot(bc_ref[...], wcbf_ref[...], preferred_element_type=jnp.float32)


def _cb(bases_cir, W_cbf):
    n = bases_cir.shape[0]
    return pl.pallas_call(
        _cb_body,
        grid=(n // BLK_T,),
        in_specs=[
            pl.BlockSpec((BLK_T, E_CBF), lambda i: (i, 0)),
            pl.BlockSpec((E_CBF, E_QIN), lambda i: (0, 0)),
        ],
        out_specs=pl.BlockSpec((BLK_T, E_QIN), lambda i: (i, 0)),
        out_shape=jax.ShapeDtypeStruct((n, E_QIN), jnp.float32),
    )(bases_cir, W_cbf)


def _stageD_body(mp_ref, ss_ref, w1r_ref, wb_ref, r8_ref, t8_ref, r32_ref,
                 t32_ref, out_ref):
    B = mp_ref.shape[0]
    f32 = jnp.float32
    smf = jnp.zeros((B, NSPH * E_QIN), f32)
    for k in range(KMAX):
        ssk = ss_ref[:, k * NSPH:(k + 1) * NSPH]
        mpk = mp_ref[:, k * E_QIN:(k + 1) * E_QIN]
        smf = smf + (jnp.dot(ssk, r8_ref[...], preferred_element_type=f32) *
                     jnp.dot(mpk, t8_ref[...], preferred_element_type=f32))
    r = jnp.zeros((B, E_SBF * E_QIN), f32)
    for n in range(NSPH):
        w1n = w1r_ref[:, n * E_SBF:(n + 1) * E_SBF]
        smn = smf[:, n * E_QIN:(n + 1) * E_QIN]
        r = r + (jnp.dot(w1n, r32_ref[...], preferred_element_type=f32) *
                 jnp.dot(smn, t32_ref[...], preferred_element_type=f32))
    out_ref[...] = jnp.dot(r, wb_ref[...], preferred_element_type=f32)


def _expander_mats():
    i8 = jnp.arange(NSPH)
    i32 = jnp.arange(E_QIN)
    j256 = jnp.arange(NSPH * E_QIN)
    j1024 = jnp.arange(E_SBF * E_QIN)
    r8 = (j256[None, :] // E_QIN == i8[:, None]).astype(jnp.float32)
    t8 = (j256[None, :] % E_QIN == i32[:, None]).astype(jnp.float32)
    r32 = (j1024[None, :] // E_QIN == i32[:, None]).astype(jnp.float32)
    t32 = (j1024[None, :] % E_QIN == i32[:, None]).astype(jnp.float32)
    return r8, t8, r32, t32


def _stageD(mp_flat, ss_flat, w1r_flat, W_bil):
    n = mp_flat.shape[0]
    r8, t8, r32, t32 = _expander_mats()
    return pl.pallas_call(
        _stageD_body,
        grid=(n // BLK_D,),
        in_specs=[
            pl.BlockSpec((BLK_D, KMAX * E_QIN), lambda i: (i, 0)),
            pl.BlockSpec((BLK_D, KMAX * NSPH), lambda i: (i, 0)),
            pl.BlockSpec((BLK_D, NSPH * E_SBF), lambda i: (i, 0)),
            pl.BlockSpec((E_SBF * E_QIN, E_QOUT), lambda i: (0, 0)),
            pl.BlockSpec((NSPH, NSPH * E_QIN), lambda i: (0, 0)),
            pl.BlockSpec((E_QIN, NSPH * E_QIN), lambda i: (0, 0)),
            pl.BlockSpec((E_QIN, E_SBF * E_QIN), lambda i: (0, 0)),
            pl.BlockSpec((E_QIN, E_SBF * E_QIN), lambda i: (0, 0)),
        ],
        out_specs=pl.BlockSpec((BLK_D, E_QOUT), lambda i: (i, 0)),
        out_shape=jax.ShapeDtypeStruct((n, E_QOUT), jnp.float32),
    )(mp_flat, ss_flat, w1r_flat, W_bil, r8, t8, r32, t32)


def _final_body(x_ref, xg_ref, wca_ref, wac_ref, out_ref):
    inv_sqrt_2 = 1.0 / math.sqrt(2.0)
    a = jnp.dot(x_ref[...], wca_ref[...], preferred_element_type=jnp.float32)
    b = jnp.dot(xg_ref[...], wac_ref[...], preferred_element_type=jnp.float32)
    out_ref[...] = (a + b) * inv_sqrt_2


def _final(x, xg, W_up_ca, W_up_ac):
    n = x.shape[0]
    return pl.pallas_call(
        _final_body,
        grid=(n // BLK_F,),
        in_specs=[
            pl.BlockSpec((BLK_F, E_QOUT), lambda i: (i, 0)),
            pl.BlockSpec((BLK_F, E_QOUT), lambda i: (i, 0)),
            pl.BlockSpec((E_QOUT, E_EDGE), lambda i: (0, 0)),
            pl.BlockSpec((E_QOUT, E_EDGE), lambda i: (0, 0)),
        ],
        out_specs=pl.BlockSpec((BLK_F, E_EDGE), lambda i: (i, 0)),
        out_shape=jax.ShapeDtypeStruct((n, E_EDGE), jnp.float32),
    )(x, xg, W_up_ca, W_up_ac)


N_QUAD = 640000
N_TRIP = 640000
NSLOTS = N_EDGES * KMAX
NW = 32
OWN = NSLOTS // NW
CHUNK = 16000
WIN = 400
SENT = 4096
ZPAD = 512

_SC_PARAMS = pltpu.CompilerParams(use_tc_tiling_on_sc=False,
                                  needs_layout_passes=False)


def _sc_mesh():
    return plsc.VectorSubcoreMesh(core_axis_name="c", subcore_axis_name="s")


def _sc_build_mpad(idx_out, idx_out_agg, ttq_ext, tii_ext, x1z, cbz):
    @functools.partial(
        pl.kernel, mesh=_sc_mesh(),
        out_type=jax.ShapeDtypeStruct((NSLOTS, E_QIN), jnp.float32),
        compiler_params=_SC_PARAMS,
        scratch_types=[
            pltpu.VMEM((OWN,), jnp.int32),
            pltpu.VMEM((CHUNK,), jnp.int32),
            pltpu.VMEM((CHUNK,), jnp.int32),
            pltpu.VMEM((WIN,), jnp.int32),
            pltpu.VMEM((WIN,), jnp.int32),
            pltpu.VMEM((WIN,), jnp.int32),
            pltpu.VMEM((WIN, E_QIN), jnp.float32),
            pltpu.VMEM((WIN, E_QIN), jnp.float32),
            pltpu.VMEM((WIN, E_QIN), jnp.float32),
            pltpu.SemaphoreType.DMA,
        ],
    )
    def k(io_hbm, ia_hbm, ttq_hbm, tii_hbm, x1_hbm, cb_hbm, out_hbm,
          winner_v, io_v, ia_v, widx_v, t_v, e2_v, x1_v, cb_v, o_v, sem):
        wid = lax.axis_index("s") * 2 + lax.axis_index("c")
        lo = wid * OWN
        iota16 = lax.iota(jnp.int32, 16)
        zero16 = iota16 - iota16

        def zi(i, c):
            winner_v[pl.ds(i * 16, 16)] = zero16
            return c
        lax.fori_loop(0, OWN // 16, zi, 0)

        def chunk_body(ci, c):
            pltpu.sync_copy(io_hbm.at[pl.ds(ci * CHUNK, CHUNK)], io_v)
            pltpu.sync_copy(ia_hbm.at[pl.ds(ci * CHUNK, CHUNK)], ia_v)
            qbase = ci * CHUNK + 1

            def vb(i, c2):
                io = io_v[pl.ds(i * 16, 16)]
                ia = ia_v[pl.ds(i * 16, 16)]
                local = io * KMAX + ia - lo
                mask = (local >= 0) & (local < OWN)
                localc = jnp.where(mask, local, 0)
                q1 = (qbase + i * 16) + iota16
                plsc.store_scatter(winner_v, [localc], q1, mask=mask)
                return c2
            lax.fori_loop(0, CHUNK // 16, vb, 0)
            return c
        lax.fori_loop(0, N_QUAD // CHUNK, chunk_body, 0)

        def win_body(wi, c):
            woff = wi * WIN

            def tb(i, c2):
                wv = winner_v[pl.ds(woff + i * 16, 16)]
                sent = N_TRIP + ((woff + i * 16 + iota16) & (SENT - 1))
                widx_v[pl.ds(i * 16, 16)] = jnp.where(wv == 0, sent, wv - 1)
                return c2
            lax.fori_loop(0, WIN // 16, tb, 0)
            pltpu.async_copy(ttq_hbm.at[widx_v], t_v, sem).wait()
            pltpu.async_copy(tii_hbm.at[t_v], e2_v, sem).wait()
            pltpu.async_copy(x1_hbm.at[e2_v], x1_v, sem).wait()
            pltpu.async_copy(cb_hbm.at[t_v], cb_v, sem).wait()

            def mb(i, c2):
                o_v[i, pl.ds(0, 16)] = x1_v[i, pl.ds(0, 16)] * cb_v[i, pl.ds(0, 16)]
                o_v[i, pl.ds(16, 16)] = x1_v[i, pl.ds(16, 16)] * cb_v[i, pl.ds(16, 16)]
                return c2
            lax.fori_loop(0, WIN, mb, 0)
            pltpu.sync_copy(o_v, out_hbm.at[pl.ds(lo + woff, WIN)])
            return c
        lax.fori_loop(0, OWN // WIN, win_body, 0)

    return k(idx_out, idx_out_agg, ttq_ext, tii_ext, x1z, cbz)


GW = N_EDGES // NW
GWIN = 1000


def _sc_gather_rows(x, idx):
    @functools.partial(
        pl.kernel, mesh=_sc_mesh(),
        out_type=jax.ShapeDtypeStruct((N_EDGES, E_QOUT), jnp.float32),
        compiler_params=_SC_PARAMS,
        scratch_types=[
            pltpu.VMEM((GWIN,), jnp.int32),
            pltpu.VMEM((GWIN, E_QOUT), jnp.float32),
            pltpu.SemaphoreType.DMA,
        ],
    )
    def k(x_hbm, idx_hbm, out_hbm, idx_v, rows_v, sem):
        wid = lax.axis_index("s") * 2 + lax.axis_index("c")
        base = wid * GW

        def wb(i, c):
            off = base + i * GWIN
            pltpu.sync_copy(idx_hbm.at[pl.ds(off, GWIN)], idx_v)
            pltpu.async_copy(x_hbm.at[idx_v], rows_v, sem).wait()
            pltpu.sync_copy(rows_v, out_hbm.at[pl.ds(off, GWIN)])
            return c
        lax.fori_loop(0, GW // GWIN, wb, 0)

    return k(x, idx)


def kernel(m, bases_rad, bases_cir, sph_rbf_W1, sph_sph, idx_triplet_in_in,
           idx_trip_in_to_quad, idx_out, idx_out_agg, id_swap,
           W_db, W_rbf, W_cbf, W_down, W_bil, W_up_ca, W_up_ac):
    x1 = _stage1(m, bases_rad, W_db, W_rbf, W_down)
    cb = _cb(bases_cir, W_cbf)
    nE = sph_rbf_W1.shape[0]
    x1z = jnp.concatenate([x1, jnp.zeros((ZPAD, E_QIN), jnp.float32)], axis=0)
    cbz = jnp.concatenate([cb, jnp.zeros((SENT, E_QIN), jnp.float32)], axis=0)
    sj = jnp.arange(SENT, dtype=jnp.int32)
    ttq_ext = jnp.concatenate(
        [idx_trip_in_to_quad.astype(jnp.int32), N_TRIP + sj])
    tii_ext = jnp.concatenate(
        [idx_triplet_in_in.astype(jnp.int32),
         N_EDGES + (sj & (ZPAD - 1))])
    mp = _sc_build_mpad(idx_out.astype(jnp.int32), idx_out_agg.astype(jnp.int32),
                        ttq_ext, tii_ext, x1z, cbz)
    mp_flat = mp.reshape(nE, KMAX * E_QIN)
    ss_flat = sph_sph.reshape(nE, KMAX * NSPH)
    w1r_flat = jnp.transpose(sph_rbf_W1, (0, 2, 1)).reshape(nE, NSPH * E_SBF)
    x = _stageD(mp_flat, ss_flat, w1r_flat, W_bil)
    xg = _sc_gather_rows(x, id_swap.astype(jnp.int32))
    return _final(x, xg, W_up_ca, W_up_ac)


# speedup vs baseline: 7.8093x; 1.0069x over previous
import math
import functools

import jax
import jax.numpy as jnp
from jax import lax
from jax.experimental import pallas as pl
from jax.experimental.pallas import tpu as pltpu, tpu_sc as plsc

N_EDGES = 160000
E_EDGE = 256
E_QIN = 32
E_SBF = 32
E_RBF = 16
E_CBF = 16
E_QOUT = 32
NSPH = 8
KMAX = 8

BLK_E = 1000
BLK_T = 2000
BLK_D = 800
BLK_F = 1000


def _stage1_body(m_ref, br_ref, wdb_ref, wrbf_ref, wdown_ref, out_ref):
    t = jnp.dot(m_ref[...], wdb_ref[...], preferred_element_type=jnp.float32)
    rb = jnp.dot(br_ref[...], wrbf_ref[...], preferred_element_type=jnp.float32)
    out_ref[...] = jnp.dot(t * rb, wdown_ref[...], preferred_element_type=jnp.float32)


def _stage1(m, bases_rad, W_db, W_rbf, W_down):
    n = m.shape[0]
    return pl.pallas_call(
        _stage1_body,
        grid=(n // BLK_E,),
        in_specs=[
            pl.BlockSpec((BLK_E, E_EDGE), lambda i: (i, 0)),
            pl.BlockSpec((BLK_E, E_RBF), lambda i: (i, 0)),
            pl.BlockSpec((E_EDGE, E_EDGE), lambda i: (0, 0)),
            pl.BlockSpec((E_RBF, E_EDGE), lambda i: (0, 0)),
            pl.BlockSpec((E_EDGE, E_QIN), lambda i: (0, 0)),
        ],
        out_specs=pl.BlockSpec((BLK_E, E_QIN), lambda i: (i, 0)),
        out_shape=jax.ShapeDtypeStruct((n, E_QIN), jnp.float32),
    )(m, bases_rad, W_db, W_rbf, W_down)


def _cb_body(bc_ref, wcbf_ref, out_ref):
    out_ref[...] = jnp.dot(bc_ref[...], wcbf_ref[...], preferred_element_type=jnp.float32)


def _cb(bases_cir, W_cbf):
    n = bases_cir.shape[0]
    return pl.pallas_call(
        _cb_body,
        grid=(n // BLK_T,),
        in_specs=[
            pl.BlockSpec((BLK_T, E_CBF), lambda i: (i, 0)),
            pl.BlockSpec((E_CBF, E_QIN), lambda i: (0, 0)),
        ],
        out_specs=pl.BlockSpec((BLK_T, E_QIN), lambda i: (i, 0)),
        out_shape=jax.ShapeDtypeStruct((n, E_QIN), jnp.float32),
    )(bases_cir, W_cbf)


def _stageD_body(mp_ref, ss_ref, w1r_ref, wb_ref, r8_ref, t8_ref, r32_ref,
                 t32_ref, out_ref):
    B = mp_ref.shape[0]
    f32 = jnp.float32
    smf = jnp.zeros((B, NSPH * E_QIN), f32)
    for k in range(KMAX):
        ssk = ss_ref[:, k * NSPH:(k + 1) * NSPH]
        mpk = mp_ref[:, k * E_QIN:(k + 1) * E_QIN]
        smf = smf + (jnp.dot(ssk, r8_ref[...], preferred_element_type=f32) *
                     jnp.dot(mpk, t8_ref[...], preferred_element_type=f32))
    r = jnp.zeros((B, E_SBF * E_QIN), f32)
    for n in range(NSPH):
        w1n = w1r_ref[:, n * E_SBF:(n + 1) * E_SBF]
        smn = smf[:, n * E_QIN:(n + 1) * E_QIN]
        r = r + (jnp.dot(w1n, r32_ref[...], preferred_element_type=f32) *
                 jnp.dot(smn, t32_ref[...], preferred_element_type=f32))
    out_ref[...] = jnp.dot(r, wb_ref[...], preferred_element_type=f32)


def _expander_mats():
    i8 = jnp.arange(NSPH)
    i32 = jnp.arange(E_QIN)
    j256 = jnp.arange(NSPH * E_QIN)
    j1024 = jnp.arange(E_SBF * E_QIN)
    r8 = (j256[None, :] // E_QIN == i8[:, None]).astype(jnp.float32)
    t8 = (j256[None, :] % E_QIN == i32[:, None]).astype(jnp.float32)
    r32 = (j1024[None, :] // E_QIN == i32[:, None]).astype(jnp.float32)
    t32 = (j1024[None, :] % E_QIN == i32[:, None]).astype(jnp.float32)
    return r8, t8, r32, t32


def _stageD(mp_flat, ss_flat, w1r_flat, W_bil):
    n = mp_flat.shape[0]
    r8, t8, r32, t32 = _expander_mats()
    return pl.pallas_call(
        _stageD_body,
        grid=(n // BLK_D,),
        in_specs=[
            pl.BlockSpec((BLK_D, KMAX * E_QIN), lambda i: (i, 0)),
            pl.BlockSpec((BLK_D, KMAX * NSPH), lambda i: (i, 0)),
            pl.BlockSpec((BLK_D, NSPH * E_SBF), lambda i: (i, 0)),
            pl.BlockSpec((E_SBF * E_QIN, E_QOUT), lambda i: (0, 0)),
            pl.BlockSpec((NSPH, NSPH * E_QIN), lambda i: (0, 0)),
            pl.BlockSpec((E_QIN, NSPH * E_QIN), lambda i: (0, 0)),
            pl.BlockSpec((E_QIN, E_SBF * E_QIN), lambda i: (0, 0)),
            pl.BlockSpec((E_QIN, E_SBF * E_QIN), lambda i: (0, 0)),
        ],
        out_specs=pl.BlockSpec((BLK_D, E_QOUT), lambda i: (i, 0)),
        out_shape=jax.ShapeDtypeStruct((n, E_QOUT), jnp.float32),
    )(mp_flat, ss_flat, w1r_flat, W_bil, r8, t8, r32, t32)


def _final_body(x_ref, xg_ref, wca_ref, wac_ref, out_ref):
    inv_sqrt_2 = 1.0 / math.sqrt(2.0)
    a = jnp.dot(x_ref[...], wca_ref[...], preferred_element_type=jnp.float32)
    b = jnp.dot(xg_ref[...], wac_ref[...], preferred_element_type=jnp.float32)
    out_ref[...] = (a + b) * inv_sqrt_2


def _final(x, xg, W_up_ca, W_up_ac):
    n = x.shape[0]
    return pl.pallas_call(
        _final_body,
        grid=(n // BLK_F,),
        in_specs=[
            pl.BlockSpec((BLK_F, E_QOUT), lambda i: (i, 0)),
            pl.BlockSpec((BLK_F, E_QOUT), lambda i: (i, 0)),
            pl.BlockSpec((E_QOUT, E_EDGE), lambda i: (0, 0)),
            pl.BlockSpec((E_QOUT, E_EDGE), lambda i: (0, 0)),
        ],
        out_specs=pl.BlockSpec((BLK_F, E_EDGE), lambda i: (i, 0)),
        out_shape=jax.ShapeDtypeStruct((n, E_EDGE), jnp.float32),
    )(x, xg, W_up_ca, W_up_ac)


N_QUAD = 640000
N_TRIP = 640000
NSLOTS = N_EDGES * KMAX
NW = 32
OWN = NSLOTS // NW
CHUNK = 16000
WIN = 400
SENT = 4096
ZPAD = 512

_SC_PARAMS = pltpu.CompilerParams(use_tc_tiling_on_sc=False,
                                  needs_layout_passes=False)


def _sc_mesh():
    return plsc.VectorSubcoreMesh(core_axis_name="c", subcore_axis_name="s")


def _sc_build_mpad(idx_out, idx_out_agg, ttq_ext, tii_ext, x1z, cbz):
    @functools.partial(
        pl.kernel, mesh=_sc_mesh(),
        out_type=jax.ShapeDtypeStruct((NSLOTS, E_QIN), jnp.float32),
        compiler_params=_SC_PARAMS,
        scratch_types=[
            pltpu.VMEM((OWN,), jnp.int32),
            pltpu.VMEM((CHUNK,), jnp.int32),
            pltpu.VMEM((CHUNK,), jnp.int32),
            pltpu.VMEM((WIN,), jnp.int32),
            pltpu.VMEM((WIN,), jnp.int32),
            pltpu.VMEM((WIN,), jnp.int32),
            pltpu.VMEM((WIN, E_QIN), jnp.float32),
            pltpu.VMEM((WIN, E_QIN), jnp.float32),
            pltpu.VMEM((WIN, E_QIN), jnp.float32),
            pltpu.SemaphoreType.DMA,
        ],
    )
    def k(io_hbm, ia_hbm, ttq_hbm, tii_hbm, x1_hbm, cb_hbm, out_hbm,
          winner_v, io_v, ia_v, widx_v, t_v, e2_v, x1_v, cb_v, o_v, sem):
        wid = lax.axis_index("s") * 2 + lax.axis_index("c")
        lo = wid * OWN
        iota16 = lax.iota(jnp.int32, 16)
        zero16 = iota16 - iota16

        def zi(i, c):
            winner_v[pl.ds(i * 16, 16)] = zero16
            return c
        lax.fori_loop(0, OWN // 16, zi, 0)

        def chunk_body(ci, c):
            pltpu.sync_copy(io_hbm.at[pl.ds(ci * CHUNK, CHUNK)], io_v)
            pltpu.sync_copy(ia_hbm.at[pl.ds(ci * CHUNK, CHUNK)], ia_v)
            qbase = ci * CHUNK + 1

            def vb(i, c2):
                for u in range(4):
                    off = i * 64 + u * 16
                    io = io_v[pl.ds(off, 16)]
                    ia = ia_v[pl.ds(off, 16)]
                    local = io * KMAX + ia - lo
                    mask = (local >= 0) & (local < OWN)
                    localc = jnp.where(mask, local, 0)
                    q1 = (qbase + off) + iota16
                    plsc.store_scatter(winner_v, [localc], q1, mask=mask)
                return c2
            lax.fori_loop(0, CHUNK // 64, vb, 0)
            return c
        lax.fori_loop(0, N_QUAD // CHUNK, chunk_body, 0)

        def win_body(wi, c):
            woff = wi * WIN

            def tb(i, c2):
                wv = winner_v[pl.ds(woff + i * 16, 16)]
                sent = N_TRIP + ((woff + i * 16 + iota16) & (SENT - 1))
                widx_v[pl.ds(i * 16, 16)] = jnp.where(wv == 0, sent, wv - 1)
                return c2
            lax.fori_loop(0, WIN // 16, tb, 0)
            pltpu.async_copy(ttq_hbm.at[widx_v], t_v, sem).wait()
            pltpu.async_copy(tii_hbm.at[t_v], e2_v, sem).wait()
            pltpu.async_copy(x1_hbm.at[e2_v], x1_v, sem).wait()
            pltpu.async_copy(cb_hbm.at[t_v], cb_v, sem).wait()

            def mb(i, c2):
                for u in range(4):
                    row = i * 4 + u
                    for h in (0, 16):
                        o_v[row, pl.ds(h, 16)] = (x1_v[row, pl.ds(h, 16)] *
                                                  cb_v[row, pl.ds(h, 16)])
                return c2
            lax.fori_loop(0, WIN // 4, mb, 0)
            pltpu.sync_copy(o_v, out_hbm.at[pl.ds(lo + woff, WIN)])
            return c
        lax.fori_loop(0, OWN // WIN, win_body, 0)

    return k(idx_out, idx_out_agg, ttq_ext, tii_ext, x1z, cbz)


GW = N_EDGES // NW
GWIN = 1000


def _sc_gather_rows(x, idx):
    @functools.partial(
        pl.kernel, mesh=_sc_mesh(),
        out_type=jax.ShapeDtypeStruct((N_EDGES, E_QOUT), jnp.float32),
        compiler_params=_SC_PARAMS,
        scratch_types=[
            pltpu.VMEM((GWIN,), jnp.int32),
            pltpu.VMEM((GWIN, E_QOUT), jnp.float32),
            pltpu.SemaphoreType.DMA,
        ],
    )
    def k(x_hbm, idx_hbm, out_hbm, idx_v, rows_v, sem):
        wid = lax.axis_index("s") * 2 + lax.axis_index("c")
        base = wid * GW

        def wb(i, c):
            off = base + i * GWIN
            pltpu.sync_copy(idx_hbm.at[pl.ds(off, GWIN)], idx_v)
            pltpu.async_copy(x_hbm.at[idx_v], rows_v, sem).wait()
            pltpu.sync_copy(rows_v, out_hbm.at[pl.ds(off, GWIN)])
            return c
        lax.fori_loop(0, GW // GWIN, wb, 0)

    return k(x, idx)


def kernel(m, bases_rad, bases_cir, sph_rbf_W1, sph_sph, idx_triplet_in_in,
           idx_trip_in_to_quad, idx_out, idx_out_agg, id_swap,
           W_db, W_rbf, W_cbf, W_down, W_bil, W_up_ca, W_up_ac):
    x1 = _stage1(m, bases_rad, W_db, W_rbf, W_down)
    cb = _cb(bases_cir, W_cbf)
    nE = sph_rbf_W1.shape[0]
    x1z = jnp.concatenate([x1, jnp.zeros((ZPAD, E_QIN), jnp.float32)], axis=0)
    cbz = jnp.concatenate([cb, jnp.zeros((SENT, E_QIN), jnp.float32)], axis=0)
    sj = jnp.arange(SENT, dtype=jnp.int32)
    ttq_ext = jnp.concatenate(
        [idx_trip_in_to_quad.astype(jnp.int32), N_TRIP + sj])
    tii_ext = jnp.concatenate(
        [idx_triplet_in_in.astype(jnp.int32),
         N_EDGES + (sj & (ZPAD - 1))])
    mp = _sc_build_mpad(idx_out.astype(jnp.int32), idx_out_agg.astype(jnp.int32),
                        ttq_ext, tii_ext, x1z, cbz)
    mp_flat = mp.reshape(nE, KMAX * E_QIN)
    ss_flat = sph_sph.reshape(nE, KMAX * NSPH)
    w1r_flat = jnp.transpose(sph_rbf_W1, (0, 2, 1)).reshape(nE, NSPH * E_SBF)
    x = _stageD(mp_flat, ss_flat, w1r_flat, W_bil)
    xg = _sc_gather_rows(x, id_swap.astype(jnp.int32))
    return _final(x, xg, W_up_ca, W_up_ac)
